# baseline (speedup 1.0000x reference)
import functools

import jax
import jax.numpy as jnp
from jax import lax
from jax.experimental import pallas as pl
from jax.experimental.pallas import tpu as pltpu
from jax.experimental.pallas import tpu_sc as plsc

DIM = 64
NWORKERS = 32
LANES = 16
NBUF = 2
EPS = 1e-5

_GDN = lax.GatherDimensionNumbers(
    offset_dims=(), collapsed_slice_dims=(0,), start_index_map=(0,))


def _lane_perm(v, idx):
    return lax.gather(v, idx[:, None], _GDN, (1,),
                      mode=lax.GatherScatterMode.PROMISE_IN_BOUNDS)


def _sc_kernel(nb, nl):
    rows_per_w = nb // NWORKERS
    subs = []
    off = 0
    while off < nl:
        n = min(128, nl - off)
        subs.append((off, n))
        off += n
    assert all(n % 8 == 0 for _, n in subs)
    gstarts = list(range(0, nl - LANES + 1, LANES))
    if gstarts[-1] + LANES < nl:
        gstarts.append(nl - LANES)

    mesh = plsc.VectorSubcoreMesh(core_axis_name="c", subcore_axis_name="s")

    @functools.partial(
        pl.kernel,
        out_type=jax.ShapeDtypeStruct((nb, nl, DIM), jnp.float32),
        mesh=mesh,
        scratch_types=[
            pltpu.VMEM((NBUF, 1, nl), jnp.int32),
            pltpu.VMEM((NBUF, 1, nl), jnp.int32),
            pltpu.VMEM((NBUF, nl, DIM), jnp.float32),
            pltpu.VMEM((NBUF, 1, nl, DIM), jnp.float32),
            pltpu.VMEM((100 * DIM,), jnp.float32),
            pltpu.VMEM((DIM,), jnp.float32),
            pltpu.VMEM((DIM,), jnp.float32),
            pltpu.SemaphoreType.DMA((NBUF,)),
            pltpu.SemaphoreType.DMA((NBUF,)),
        ],
        compiler_params=pltpu.CompilerParams(
            needs_layout_passes=False, use_tc_tiling_on_sc=False),
    )
    def kern(widx_hbm, pidx_hbm, table_hbm, pos_hbm, gam_hbm, bet_hbm,
             out_hbm, widx_v, pidx_v, xrows_v, outbuf_v, pos_v,
             gam_v, bet_v, gsem, osem):
        wid = lax.axis_index("s") * 2 + lax.axis_index("c")
        wrow = wid * rows_per_w

        pltpu.sync_copy(pos_hbm, pos_v)
        pltpu.sync_copy(gam_hbm, gam_v)
        pltpu.sync_copy(bet_hbm, bet_v)
        gb = [(gam_v[pl.ds(k * LANES, LANES)], bet_v[pl.ds(k * LANES, LANES)])
              for k in range(DIM // LANES)]
        iota16 = lax.iota(jnp.int32, LANES)
        perms = [jnp.bitwise_xor(iota16, jnp.int32(d)) for d in (1, 2, 4, 8)]
        masks = [jnp.bitwise_and(iota16, jnp.int32(d)) == 0
                 for d in (1, 2, 4, 8)]

        def start_chunk(c, slot):
            row = wrow + c
            pltpu.sync_copy(widx_hbm.at[pl.ds(row, 1)], widx_v.at[slot])
            pltpu.sync_copy(pidx_hbm.at[pl.ds(row, 1)], pidx_v.at[slot])
            for off, n in subs:
                pltpu.async_copy(
                    table_hbm.at[widx_v.at[slot, 0, pl.ds(off, n)]],
                    xrows_v.at[slot, pl.ds(off, n)], gsem.at[slot])

        for c in range(NBUF):
            start_chunk(c, c)

        def fold(a, b, mask, pr):
            return (jnp.where(mask, a, _lane_perm(b, pr))
                    + jnp.where(mask, _lane_perm(a, pr), b))

        def chunk_body(c, carry):
            slot = lax.rem(c, NBUF)
            row = wrow + c
            for off, n in subs:
                pltpu.make_async_copy(
                    table_hbm.at[widx_v.at[slot, 0, pl.ds(off, n)]],
                    xrows_v.at[slot, pl.ds(off, n)],
                    gsem.at[slot]).wait()

            @pl.when(c >= NBUF)
            def _():
                pltpu.make_async_copy(
                    outbuf_v.at[slot],
                    out_hbm.at[pl.ds(row - NBUF, 1)], osem.at[slot]).wait()

            def group_body(start):
                pidv = pidx_v[slot, 0, pl.ds(start, LANES)]
                for j0 in range(0, LANES, 4):
                    hs, ss, qs = [], [], []
                    for j in range(4):
                        i = start + j0 + j
                        pbase = pidv[j0 + j] * DIM
                        h = []
                        for k in range(DIM // LANES):
                            x = xrows_v[slot, i, pl.ds(k * LANES, LANES)]
                            p = pos_v[pl.ds(pbase + k * LANES, LANES)]
                            h.append(x + p)
                        hs.append(h)
                        ss.append((h[0] + h[1]) + (h[2] + h[3]))
                        qs.append((h[0] * h[0] + h[1] * h[1])
                                  + (h[2] * h[2] + h[3] * h[3]))
                    sm = fold(fold(ss[0], ss[1], masks[0], perms[0]),
                              fold(ss[2], ss[3], masks[0], perms[0]),
                              masks[1], perms[1])
                    qm = fold(fold(qs[0], qs[1], masks[0], perms[0]),
                              fold(qs[2], qs[3], masks[0], perms[0]),
                              masks[1], perms[1])
                    sm = sm + _lane_perm(sm, perms[2])
                    sm = sm + _lane_perm(sm, perms[3])
                    qm = qm + _lane_perm(qm, perms[2])
                    qm = qm + _lane_perm(qm, perms[3])
                    meanv = sm * (1.0 / DIM)
                    av = qm * (1.0 / DIM) - meanv * meanv + EPS
                    ib = jnp.int32(0x5F3759DF) - jnp.right_shift(
                        plsc.bitcast(av, jnp.int32), 1)
                    y = plsc.bitcast(ib, jnp.float32)
                    for _unused in range(2):
                        y = y * (1.5 - 0.5 * av * y * y)
                    ms = meanv * y
                    for j in range(4):
                        i = start + j0 + j
                        pj = jnp.full((LANES,), j, jnp.int32)
                        ysp = _lane_perm(y, pj)
                        mssp = _lane_perm(ms, pj)
                        for k in range(DIM // LANES):
                            gk, bk = gb[k]
                            outbuf_v[slot, 0, i, pl.ds(k * LANES, LANES)] = (
                                (hs[j][k] * ysp - mssp) * gk + bk)

            for start in gstarts:
                group_body(start)

            pltpu.async_copy(outbuf_v.at[slot],
                             out_hbm.at[pl.ds(row, 1)], osem.at[slot])

            @pl.when(c + NBUF < rows_per_w)
            def _():
                start_chunk(c + NBUF, slot)
            return carry

        lax.fori_loop(0, rows_per_w, chunk_body, 0)

        for c in range(rows_per_w - NBUF, rows_per_w):
            slot = c % NBUF
            pltpu.make_async_copy(
                outbuf_v.at[slot],
                out_hbm.at[pl.ds(wrow + c, 1)], osem.at[slot]).wait()

    return kern


def kernel(tcword_id, position_ids, table, pos_embs, gamma, beta):
    b, l = tcword_id.shape
    return _sc_kernel(b, l)(
        tcword_id.astype(jnp.int32), position_ids.astype(jnp.int32),
        table, pos_embs.reshape(100 * DIM), gamma, beta)


# speedup vs baseline: 1.6435x; 1.6435x over previous
import functools

import jax
import jax.numpy as jnp
from jax import lax
from jax.experimental import pallas as pl
from jax.experimental.pallas import tpu as pltpu
from jax.experimental.pallas import tpu_sc as plsc

DIM = 64
NWORKERS = 32
LANES = 16
NBUF = 2
EPS = 1e-5

_GDN = lax.GatherDimensionNumbers(
    offset_dims=(), collapsed_slice_dims=(0,), start_index_map=(0,))


def _lane_perm(v, idx):
    return lax.gather(v, idx[:, None], _GDN, (1,),
                      mode=lax.GatherScatterMode.PROMISE_IN_BOUNDS)


def _sc_kernel(nb, nl):
    rows_per_w = nb // NWORKERS
    subs = []
    off = 0
    while off < nl:
        n = min(128, nl - off)
        subs.append((off, n))
        off += n
    assert all(n % 8 == 0 for _, n in subs)
    gstarts = list(range(0, nl - LANES + 1, LANES))
    if gstarts[-1] + LANES < nl:
        gstarts.append(nl - LANES)

    mesh = plsc.VectorSubcoreMesh(core_axis_name="c", subcore_axis_name="s")

    @functools.partial(
        pl.kernel,
        out_type=jax.ShapeDtypeStruct((nb, nl, DIM), jnp.float32),
        mesh=mesh,
        scratch_types=[
            pltpu.VMEM((NBUF, 1, nl), jnp.int32),
            pltpu.VMEM((NBUF, 1, nl), jnp.int32),
            pltpu.VMEM((NBUF, nl, DIM), jnp.float32),
            pltpu.VMEM((NBUF, 1, nl, DIM), jnp.float32),
            pltpu.VMEM((100 * DIM,), jnp.float32),
            pltpu.VMEM((DIM,), jnp.float32),
            pltpu.VMEM((DIM,), jnp.float32),
            pltpu.SemaphoreType.DMA((NBUF,)),
            pltpu.SemaphoreType.DMA((NBUF,)),
        ],
        compiler_params=pltpu.CompilerParams(
            needs_layout_passes=False, use_tc_tiling_on_sc=False),
    )
    def kern(widx_hbm, pidx_hbm, table_hbm, pos_hbm, gam_hbm, bet_hbm,
             out_hbm, widx_v, pidx_v, xrows_v, outbuf_v, pos_v,
             gam_v, bet_v, gsem, osem):
        wid = lax.axis_index("s") * 2 + lax.axis_index("c")
        wrow = wid * rows_per_w

        pltpu.sync_copy(pos_hbm, pos_v)
        pltpu.sync_copy(gam_hbm, gam_v)
        pltpu.sync_copy(bet_hbm, bet_v)
        gb = [(gam_v[pl.ds(k * LANES, LANES)], bet_v[pl.ds(k * LANES, LANES)])
              for k in range(DIM // LANES)]
        iota16 = lax.iota(jnp.int32, LANES)
        perms = [jnp.bitwise_xor(iota16, jnp.int32(d)) for d in (1, 2, 4, 8)]
        masks = [jnp.bitwise_and(iota16, jnp.int32(d)) == 0
                 for d in (1, 2, 4, 8)]

        def start_chunk(c, slot):
            row = wrow + c
            pltpu.sync_copy(widx_hbm.at[pl.ds(row, 1)], widx_v.at[slot])
            pltpu.sync_copy(pidx_hbm.at[pl.ds(row, 1)], pidx_v.at[slot])
            for off, n in subs:
                pltpu.async_copy(
                    table_hbm.at[widx_v.at[slot, 0, pl.ds(off, n)]],
                    xrows_v.at[slot, pl.ds(off, n)], gsem.at[slot])

        for c in range(NBUF):
            start_chunk(c, c)

        def fold(a, b, mask, pr):
            return (jnp.where(mask, a, _lane_perm(b, pr))
                    + jnp.where(mask, _lane_perm(a, pr), b))

        def chunk_body(c, carry):
            slot = lax.rem(c, NBUF)
            row = wrow + c
            for off, n in subs:
                pltpu.make_async_copy(
                    table_hbm.at[widx_v.at[slot, 0, pl.ds(off, n)]],
                    xrows_v.at[slot, pl.ds(off, n)],
                    gsem.at[slot]).wait()

            @pl.when(c >= NBUF)
            def _():
                pltpu.make_async_copy(
                    outbuf_v.at[slot],
                    out_hbm.at[pl.ds(row - NBUF, 1)], osem.at[slot]).wait()

            def group_body(g, carry2):
                start = jnp.minimum(g * LANES, jnp.int32(nl - LANES))
                pidv = pidx_v[slot, 0, pl.ds(start, LANES)]
                for j0 in range(0, LANES, 4):
                    hs, ss, qs = [], [], []
                    for j in range(4):
                        i = start + j0 + j
                        pbase = pidv[j0 + j] * DIM
                        h = []
                        for k in range(DIM // LANES):
                            x = xrows_v[slot, i, pl.ds(k * LANES, LANES)]
                            p = pos_v[pl.ds(pbase + k * LANES, LANES)]
                            h.append(x + p)
                        hs.append(h)
                        ss.append((h[0] + h[1]) + (h[2] + h[3]))
                        qs.append((h[0] * h[0] + h[1] * h[1])
                                  + (h[2] * h[2] + h[3] * h[3]))
                    sm = fold(fold(ss[0], ss[1], masks[0], perms[0]),
                              fold(ss[2], ss[3], masks[0], perms[0]),
                              masks[1], perms[1])
                    qm = fold(fold(qs[0], qs[1], masks[0], perms[0]),
                              fold(qs[2], qs[3], masks[0], perms[0]),
                              masks[1], perms[1])
                    sm = sm + _lane_perm(sm, perms[2])
                    sm = sm + _lane_perm(sm, perms[3])
                    qm = qm + _lane_perm(qm, perms[2])
                    qm = qm + _lane_perm(qm, perms[3])
                    meanv = sm * (1.0 / DIM)
                    av = qm * (1.0 / DIM) - meanv * meanv + EPS
                    ib = jnp.int32(0x5F3759DF) - jnp.right_shift(
                        plsc.bitcast(av, jnp.int32), 1)
                    y = plsc.bitcast(ib, jnp.float32)
                    for _unused in range(2):
                        y = y * (1.5 - 0.5 * av * y * y)
                    ms = meanv * y
                    for j in range(4):
                        i = start + j0 + j
                        pj = jnp.full((LANES,), j, jnp.int32)
                        ysp = _lane_perm(y, pj)
                        mssp = _lane_perm(ms, pj)
                        for k in range(DIM // LANES):
                            gk, bk = gb[k]
                            outbuf_v[slot, 0, i, pl.ds(k * LANES, LANES)] = (
                                (hs[j][k] * ysp - mssp) * gk + bk)

                return carry2

            lax.fori_loop(0, len(gstarts), group_body, 0)

            pltpu.async_copy(outbuf_v.at[slot],
                             out_hbm.at[pl.ds(row, 1)], osem.at[slot])

            @pl.when(c + NBUF < rows_per_w)
            def _():
                start_chunk(c + NBUF, slot)
            return carry

        lax.fori_loop(0, rows_per_w, chunk_body, 0)

        for c in range(rows_per_w - NBUF, rows_per_w):
            slot = c % NBUF
            pltpu.make_async_copy(
                outbuf_v.at[slot],
                out_hbm.at[pl.ds(wrow + c, 1)], osem.at[slot]).wait()

    return kern


def kernel(tcword_id, position_ids, table, pos_embs, gamma, beta):
    b, l = tcword_id.shape
    return _sc_kernel(b, l)(
        tcword_id.astype(jnp.int32), position_ids.astype(jnp.int32),
        table, pos_embs.reshape(100 * DIM), gamma, beta)
